# Initial kernel scaffold; baseline (speedup 1.0000x reference)
import jax
import jax.numpy as jnp
from jax.experimental import pallas as pl


def kernel(x, edge_index, pagerank, W0, W1):
    raise NotImplementedError("write your pallas kernel here")


# speedup vs baseline: 18.8756x; 18.8756x over previous
import functools

import jax
import jax.numpy as jnp
from jax import lax
from jax.experimental import pallas as pl
from jax.experimental.pallas import tpu as pltpu
from jax.experimental.pallas import tpu_sc as plsc

_N = 10000
_E = 320000
_D_IN = 128
_HID = 64
_DIM = 32
_NOISE_STD = 0.1
_EPS = 1e-6

_G = 125
_GROUPS = _E // _G
_NC = 2
_NS = 16
_NTILES = _NC * _NS
_GP_TILE = _GROUPS // _NTILES
_RPS = _N // _NS
_CNTW = 8

_ROWBLK = 200


def _sc_mesh():
    return plsc.VectorSubcoreMesh(core_axis_name="c", subcore_axis_name="s")


def _make_count():
    @functools.partial(
        pl.kernel,
        out_type=jax.ShapeDtypeStruct((_NC, _N, _CNTW), jnp.float32),
        mesh=_sc_mesh(),
        compiler_params=pltpu.CompilerParams(use_tc_tiling_on_sc=False),
        scratch_types=[
            pltpu.VMEM((_GP_TILE, _G), jnp.int32),
            pltpu.VMEM((_G, _CNTW), jnp.float32),
            pltpu.VMEM_SHARED((_N, _CNTW), jnp.float32),
        ],
    )
    def count_kernel(dst_hbm, ones_hbm, zeros_hbm, out_hbm, dst_v, ones_v, acc):
        c = lax.axis_index("c")
        s = lax.axis_index("s")
        wid = c * _NS + s
        pltpu.sync_copy(zeros_hbm.at[pl.ds(s * _RPS, _RPS)],
                        acc.at[pl.ds(s * _RPS, _RPS)])
        pltpu.sync_copy(dst_hbm.at[pl.ds(wid * _GP_TILE, _GP_TILE)], dst_v)
        pltpu.sync_copy(ones_hbm, ones_v)
        plsc.subcore_barrier()

        def body(j, carry):
            pltpu.sync_copy(ones_v, acc.at[dst_v.at[j]], add=True)
            return carry

        lax.fori_loop(0, _GP_TILE, body, 0)
        plsc.subcore_barrier()
        pltpu.sync_copy(acc.at[pl.ds(s * _RPS, _RPS)],
                        out_hbm.at[c, pl.ds(s * _RPS, _RPS)])

    return count_kernel


def _make_scatter(feat):
    @functools.partial(
        pl.kernel,
        out_type=jax.ShapeDtypeStruct((_NC, _N, feat), jnp.float32),
        mesh=_sc_mesh(),
        compiler_params=pltpu.CompilerParams(use_tc_tiling_on_sc=False),
        scratch_types=[
            pltpu.VMEM((_GP_TILE, _G), jnp.int32),
            pltpu.VMEM((_GP_TILE, _G), jnp.int32),
            pltpu.VMEM((_G, feat), jnp.float32),
            pltpu.VMEM_SHARED((_N, feat), jnp.float32),
            pltpu.SemaphoreType.DMA,
        ],
    )
    def scatter_kernel(src_hbm, dst_hbm, h_hbm, zeros_hbm, out_hbm,
                       src_v, dst_v, rows_v, acc, sem):
        c = lax.axis_index("c")
        s = lax.axis_index("s")
        wid = c * _NS + s
        pltpu.sync_copy(zeros_hbm.at[pl.ds(s * _RPS, _RPS)],
                        acc.at[pl.ds(s * _RPS, _RPS)])
        pltpu.sync_copy(src_hbm.at[pl.ds(wid * _GP_TILE, _GP_TILE)], src_v)
        pltpu.sync_copy(dst_hbm.at[pl.ds(wid * _GP_TILE, _GP_TILE)], dst_v)
        plsc.subcore_barrier()

        def body(j, carry):
            pltpu.async_copy(h_hbm.at[src_v.at[j]], rows_v, sem).wait()
            pltpu.sync_copy(rows_v, acc.at[dst_v.at[j]], add=True)
            return carry

        lax.fori_loop(0, _GP_TILE, body, 0)
        plsc.subcore_barrier()
        pltpu.sync_copy(acc.at[pl.ds(s * _RPS, _RPS)],
                        out_hbm.at[c, pl.ds(s * _RPS, _RPS)])

    return scatter_kernel


_count_call = _make_count()
_scatter64_call = _make_scatter(_HID)
_scatter32_call = _make_scatter(_DIM)


def _enc1_kernel(x_ref, w0_ref, cnt_ref, h1p_ref, dinv_ref):
    cnt = (jnp.sum(cnt_ref[0], axis=1, keepdims=True)
           + jnp.sum(cnt_ref[1], axis=1, keepdims=True)) * (1.0 / _CNTW)
    dinv = lax.rsqrt(cnt + 1.0)
    xw = jnp.dot(x_ref[...], w0_ref[...],
                 preferred_element_type=jnp.float32)
    h1p_ref[...] = xw * dinv
    dinv_ref[...] = dinv


def _enc2_kernel(p_ref, h1p_ref, dinv_ref, noise_ref, w1_ref, h2p_ref):
    t = (p_ref[0] + p_ref[1] + h1p_ref[...]) * dinv_ref[...]
    hidden = jnp.maximum(t, 0.0) + noise_ref[...]
    h2 = jnp.dot(hidden, w1_ref[...],
                 preferred_element_type=jnp.float32)
    h2p_ref[...] = h2 * dinv_ref[...]


def _dec_prep_kernel(q_ref, h2p_ref, dinv_ref, pr_ref, coords_ref, sq_ref, mass_ref):
    z = (q_ref[0] + q_ref[1] + h2p_ref[...]) * dinv_ref[...]
    mass_ref[...] = z[:, _DIM - 1:_DIM] + pr_ref[...]
    lane = lax.broadcasted_iota(jnp.int32, (_N, _DIM), 1)
    coords = jnp.where(lane < _DIM - 1, z, 0.0)
    nrm = jnp.sqrt(jnp.sum(coords * coords, axis=1, keepdims=True))
    cn = coords / (nrm + _EPS)
    coords_ref[...] = cn
    sq_ref[...] = jnp.sum(cn * cn, axis=1, keepdims=True)


def _dec_kernel(a_ref, call_ref, sqr_ref, sqc_ref, mass_ref, out_ref):
    dot = lax.dot_general(a_ref[...], call_ref[...],
                          (((1,), (1,)), ((), ())),
                          preferred_element_type=jnp.float32)
    d2 = jnp.maximum(sqr_ref[...] + sqc_ref[...] - 2.0 * dot, 0.0)
    out_ref[...] = mass_ref[...] - jnp.log(d2 + _EPS)


def kernel(x, edge_index, pagerank, W0, W1):
    src = edge_index[0].astype(jnp.int32).reshape(_GROUPS, _G)
    dst = edge_index[1].astype(jnp.int32).reshape(_GROUPS, _G)
    ones_rows = jnp.ones((_G, _CNTW), jnp.float32)
    zeros_cnt = jnp.zeros((_N, _CNTW), jnp.float32)
    zeros64 = jnp.zeros((_N, _HID), jnp.float32)
    zeros32 = jnp.zeros((_N, _DIM), jnp.float32)
    noise = _NOISE_STD * jax.random.normal(jax.random.key(42), (_N, _HID),
                                           dtype=jnp.float32)

    cnt = _count_call(dst, ones_rows, zeros_cnt)

    h1p, dinv = pl.pallas_call(
        _enc1_kernel,
        out_shape=[
            jax.ShapeDtypeStruct((_N, _HID), jnp.float32),
            jax.ShapeDtypeStruct((_N, 1), jnp.float32),
        ],
    )(x, W0, cnt)

    p = _scatter64_call(src, dst, h1p, zeros64)

    h2p = pl.pallas_call(
        _enc2_kernel,
        out_shape=jax.ShapeDtypeStruct((_N, _DIM), jnp.float32),
    )(p, h1p, dinv, noise, W1)

    q = _scatter32_call(src, dst, h2p, zeros32)

    coords, sq, mass = pl.pallas_call(
        _dec_prep_kernel,
        out_shape=[
            jax.ShapeDtypeStruct((_N, _DIM), jnp.float32),
            jax.ShapeDtypeStruct((_N, 1), jnp.float32),
            jax.ShapeDtypeStruct((_N, 1), jnp.float32),
        ],
    )(q, h2p, dinv, pagerank.reshape(_N, 1))

    sq_row = sq.reshape(1, _N)
    mass_row = mass.reshape(1, _N)

    logits = pl.pallas_call(
        _dec_kernel,
        grid=(_N // _ROWBLK,),
        in_specs=[
            pl.BlockSpec((_ROWBLK, _DIM), lambda i: (i, 0)),
            pl.BlockSpec((_N, _DIM), lambda i: (0, 0)),
            pl.BlockSpec((_ROWBLK, 1), lambda i: (i, 0)),
            pl.BlockSpec((1, _N), lambda i: (0, 0)),
            pl.BlockSpec((1, _N), lambda i: (0, 0)),
        ],
        out_specs=pl.BlockSpec((_ROWBLK, _N), lambda i: (i, 0)),
        out_shape=jax.ShapeDtypeStruct((_N, _N), jnp.float32),
    )(coords, coords, sq, sq_row, mass_row)
    return logits


# speedup vs baseline: 22.7766x; 1.2067x over previous
import functools

import jax
import jax.numpy as jnp
from jax import lax
from jax.experimental import pallas as pl
from jax.experimental.pallas import tpu as pltpu
from jax.experimental.pallas import tpu_sc as plsc

_N = 10000
_E = 320000
_D_IN = 128
_HID = 64
_DIM = 32
_NOISE_STD = 0.1
_EPS = 1e-6

_G = 125
_GROUPS = _E // _G
_NC = 2
_NS = 16
_NTILES = _NC * _NS
_GP_TILE = _GROUPS // _NTILES
_RPS = _N // _NS
_CNTW = 8

_ROWBLK = 200


def _sc_mesh():
    return plsc.VectorSubcoreMesh(core_axis_name="c", subcore_axis_name="s")


def _make_count():
    @functools.partial(
        pl.kernel,
        out_type=jax.ShapeDtypeStruct((_NC, _N, _CNTW), jnp.float32),
        mesh=_sc_mesh(),
        compiler_params=pltpu.CompilerParams(use_tc_tiling_on_sc=False),
        scratch_types=[
            pltpu.VMEM((_GP_TILE, _G), jnp.int32),
            pltpu.VMEM((_G, _CNTW), jnp.float32),
            pltpu.VMEM_SHARED((_N, _CNTW), jnp.float32),
        ],
    )
    def count_kernel(dst_hbm, ones_hbm, zeros_hbm, out_hbm, dst_v, ones_v, acc):
        c = lax.axis_index("c")
        s = lax.axis_index("s")
        wid = c * _NS + s
        pltpu.sync_copy(zeros_hbm.at[pl.ds(s * _RPS, _RPS)],
                        acc.at[pl.ds(s * _RPS, _RPS)])
        pltpu.sync_copy(dst_hbm.at[pl.ds(wid * _GP_TILE, _GP_TILE)], dst_v)
        pltpu.sync_copy(ones_hbm, ones_v)
        plsc.subcore_barrier()

        def body(j, carry):
            pltpu.sync_copy(ones_v, acc.at[dst_v.at[j]], add=True)
            return carry

        lax.fori_loop(0, _GP_TILE, body, 0)
        plsc.subcore_barrier()
        pltpu.sync_copy(acc.at[pl.ds(s * _RPS, _RPS)],
                        out_hbm.at[c, pl.ds(s * _RPS, _RPS)])

    return count_kernel


def _make_scatter(feat):
    @functools.partial(
        pl.kernel,
        out_type=jax.ShapeDtypeStruct((_NC, _N, feat), jnp.float32),
        mesh=_sc_mesh(),
        compiler_params=pltpu.CompilerParams(use_tc_tiling_on_sc=False),
        scratch_types=[
            pltpu.VMEM((_GP_TILE, _G), jnp.int32),
            pltpu.VMEM((_GP_TILE, _G), jnp.int32),
            pltpu.VMEM((_G, feat), jnp.float32),
            pltpu.VMEM((_G, feat), jnp.float32),
            pltpu.VMEM_SHARED((_N, feat), jnp.float32),
            pltpu.SemaphoreType.DMA,
            pltpu.SemaphoreType.DMA,
        ],
    )
    def scatter_kernel(src_hbm, dst_hbm, h_hbm, zeros_hbm, out_hbm,
                       src_v, dst_v, rows0, rows1, acc, sem0, sem1):
        c = lax.axis_index("c")
        s = lax.axis_index("s")
        wid = c * _NS + s
        pltpu.sync_copy(zeros_hbm.at[pl.ds(s * _RPS, _RPS)],
                        acc.at[pl.ds(s * _RPS, _RPS)])
        pltpu.sync_copy(src_hbm.at[pl.ds(wid * _GP_TILE, _GP_TILE)], src_v)
        pltpu.sync_copy(dst_hbm.at[pl.ds(wid * _GP_TILE, _GP_TILE)], dst_v)
        plsc.subcore_barrier()

        pltpu.async_copy(h_hbm.at[src_v.at[0]], rows0, sem0)
        pltpu.async_copy(h_hbm.at[src_v.at[1]], rows1, sem1)

        def body(i, carry):
            for b, (rows, sem) in enumerate(((rows0, sem0), (rows1, sem1))):
                j = i * 2 + b
                pltpu.make_async_copy(h_hbm.at[pl.ds(0, _G)], rows, sem).wait()
                pltpu.sync_copy(rows, acc.at[dst_v.at[j]], add=True)

                @pl.when(j + 2 < _GP_TILE)
                def _():
                    pltpu.async_copy(h_hbm.at[src_v.at[j + 2]], rows, sem)

            return carry

        lax.fori_loop(0, _GP_TILE // 2, body, 0)
        plsc.subcore_barrier()
        pltpu.sync_copy(acc.at[pl.ds(s * _RPS, _RPS)],
                        out_hbm.at[c, pl.ds(s * _RPS, _RPS)])

    return scatter_kernel


_count_call = _make_count()
_scatter64_call = _make_scatter(_HID)
_scatter32_call = _make_scatter(_DIM)


def _enc1_kernel(x_ref, w0_ref, cnt_ref, h1p_ref, dinv_ref):
    cnt = (jnp.sum(cnt_ref[0], axis=1, keepdims=True)
           + jnp.sum(cnt_ref[1], axis=1, keepdims=True)) * (1.0 / _CNTW)
    dinv = lax.rsqrt(cnt + 1.0)
    xw = jnp.dot(x_ref[...], w0_ref[...],
                 preferred_element_type=jnp.float32)
    h1p_ref[...] = xw * dinv
    dinv_ref[...] = dinv


def _enc2_kernel(p_ref, h1p_ref, dinv_ref, noise_ref, w1_ref, h2p_ref):
    t = (p_ref[0] + p_ref[1] + h1p_ref[...]) * dinv_ref[...]
    hidden = jnp.maximum(t, 0.0) + noise_ref[...]
    h2 = jnp.dot(hidden, w1_ref[...],
                 preferred_element_type=jnp.float32)
    h2p_ref[...] = h2 * dinv_ref[...]


def _dec_prep_kernel(q_ref, h2p_ref, dinv_ref, pr_ref, coords_ref, sq_ref, mass_ref):
    z = (q_ref[0] + q_ref[1] + h2p_ref[...]) * dinv_ref[...]
    mass_ref[...] = z[:, _DIM - 1:_DIM] + pr_ref[...]
    lane = lax.broadcasted_iota(jnp.int32, (_N, _DIM), 1)
    coords = jnp.where(lane < _DIM - 1, z, 0.0)
    nrm = jnp.sqrt(jnp.sum(coords * coords, axis=1, keepdims=True))
    cn = coords / (nrm + _EPS)
    coords_ref[...] = cn
    sq_ref[...] = jnp.sum(cn * cn, axis=1, keepdims=True)


def _dec_kernel(a_ref, call_ref, sqr_ref, sqc_ref, mass_ref, out_ref):
    dot = lax.dot_general(a_ref[...], call_ref[...],
                          (((1,), (1,)), ((), ())),
                          preferred_element_type=jnp.float32)
    d2 = jnp.maximum(sqr_ref[...] + sqc_ref[...] - 2.0 * dot, 0.0)
    out_ref[...] = mass_ref[...] - jnp.log(d2 + _EPS)


def kernel(x, edge_index, pagerank, W0, W1):
    src = edge_index[0].astype(jnp.int32).reshape(_GROUPS, _G)
    dst = edge_index[1].astype(jnp.int32).reshape(_GROUPS, _G)
    ones_rows = jnp.ones((_G, _CNTW), jnp.float32)
    zeros_cnt = jnp.zeros((_N, _CNTW), jnp.float32)
    zeros64 = jnp.zeros((_N, _HID), jnp.float32)
    zeros32 = jnp.zeros((_N, _DIM), jnp.float32)
    noise = _NOISE_STD * jax.random.normal(jax.random.key(42), (_N, _HID),
                                           dtype=jnp.float32)

    cnt = _count_call(dst, ones_rows, zeros_cnt)

    h1p, dinv = pl.pallas_call(
        _enc1_kernel,
        out_shape=[
            jax.ShapeDtypeStruct((_N, _HID), jnp.float32),
            jax.ShapeDtypeStruct((_N, 1), jnp.float32),
        ],
    )(x, W0, cnt)

    p = _scatter64_call(src, dst, h1p, zeros64)

    h2p = pl.pallas_call(
        _enc2_kernel,
        out_shape=jax.ShapeDtypeStruct((_N, _DIM), jnp.float32),
    )(p, h1p, dinv, noise, W1)

    q = _scatter32_call(src, dst, h2p, zeros32)

    coords, sq, mass = pl.pallas_call(
        _dec_prep_kernel,
        out_shape=[
            jax.ShapeDtypeStruct((_N, _DIM), jnp.float32),
            jax.ShapeDtypeStruct((_N, 1), jnp.float32),
            jax.ShapeDtypeStruct((_N, 1), jnp.float32),
        ],
    )(q, h2p, dinv, pagerank.reshape(_N, 1))

    sq_row = sq.reshape(1, _N)
    mass_row = mass.reshape(1, _N)

    logits = pl.pallas_call(
        _dec_kernel,
        grid=(_N // _ROWBLK,),
        in_specs=[
            pl.BlockSpec((_ROWBLK, _DIM), lambda i: (i, 0)),
            pl.BlockSpec((_N, _DIM), lambda i: (0, 0)),
            pl.BlockSpec((_ROWBLK, 1), lambda i: (i, 0)),
            pl.BlockSpec((1, _N), lambda i: (0, 0)),
            pl.BlockSpec((1, _N), lambda i: (0, 0)),
        ],
        out_specs=pl.BlockSpec((_ROWBLK, _N), lambda i: (i, 0)),
        out_shape=jax.ShapeDtypeStruct((_N, _N), jnp.float32),
    )(coords, coords, sq, sq_row, mass_row)
    return logits


# speedup vs baseline: 24.7516x; 1.0867x over previous
import functools

import jax
import jax.numpy as jnp
from jax import lax
from jax.experimental import pallas as pl
from jax.experimental.pallas import tpu as pltpu
from jax.experimental.pallas import tpu_sc as plsc

_N = 10000
_E = 320000
_D_IN = 128
_HID = 64
_DIM = 32
_NOISE_STD = 0.1
_EPS = 1e-6

_G = 125
_GROUPS = _E // _G
_NC = 2
_NS = 16
_NTILES = _NC * _NS
_GP_TILE = _GROUPS // _NTILES
_RPS = _N // _NS
_CNTW = 8

_ROWBLK = 200


def _sc_mesh():
    return plsc.VectorSubcoreMesh(core_axis_name="c", subcore_axis_name="s")


def _make_count():
    @functools.partial(
        pl.kernel,
        out_type=jax.ShapeDtypeStruct((_NC, _N, _CNTW), jnp.float32),
        mesh=_sc_mesh(),
        compiler_params=pltpu.CompilerParams(use_tc_tiling_on_sc=False),
        scratch_types=[
            pltpu.VMEM((_GP_TILE, _G), jnp.int32),
            pltpu.VMEM((_G, _CNTW), jnp.float32),
            pltpu.VMEM_SHARED((_N, _CNTW), jnp.float32),
        ],
    )
    def count_kernel(dst_hbm, ones_hbm, zeros_hbm, out_hbm, dst_v, ones_v, acc):
        c = lax.axis_index("c")
        s = lax.axis_index("s")
        wid = c * _NS + s
        pltpu.sync_copy(zeros_hbm.at[pl.ds(s * _RPS, _RPS)],
                        acc.at[pl.ds(s * _RPS, _RPS)])
        pltpu.sync_copy(dst_hbm.at[pl.ds(wid * _GP_TILE, _GP_TILE)], dst_v)
        pltpu.sync_copy(ones_hbm, ones_v)
        plsc.subcore_barrier()

        def body(j, carry):
            pltpu.sync_copy(ones_v, acc.at[dst_v.at[j]], add=True)
            return carry

        lax.fori_loop(0, _GP_TILE, body, 0)
        plsc.subcore_barrier()
        pltpu.sync_copy(acc.at[pl.ds(s * _RPS, _RPS)],
                        out_hbm.at[c, pl.ds(s * _RPS, _RPS)])

    return count_kernel


def _make_scatter(feat):
    @functools.partial(
        pl.kernel,
        out_type=jax.ShapeDtypeStruct((_NC, _N, feat), jnp.float32),
        mesh=_sc_mesh(),
        compiler_params=pltpu.CompilerParams(use_tc_tiling_on_sc=False),
        scratch_types=[
            pltpu.VMEM((_GP_TILE, _G), jnp.int32),
            pltpu.VMEM((_GP_TILE, _G), jnp.int32),
            pltpu.VMEM((_G, feat), jnp.float32),
            pltpu.VMEM((_G, feat), jnp.float32),
            pltpu.VMEM((_G, feat), jnp.float32),
            pltpu.VMEM((_G, feat), jnp.float32),
            pltpu.VMEM_SHARED((_N, feat), jnp.float32),
            pltpu.SemaphoreType.DMA,
            pltpu.SemaphoreType.DMA,
            pltpu.SemaphoreType.DMA,
            pltpu.SemaphoreType.DMA,
        ],
    )
    def scatter_kernel(src_hbm, dst_hbm, h_hbm, zeros_hbm, out_hbm,
                       src_v, dst_v, rows0, rows1, rows2, rows3,
                       acc, sem0, sem1, sem2, sem3):
        c = lax.axis_index("c")
        s = lax.axis_index("s")
        wid = c * _NS + s
        pltpu.sync_copy(zeros_hbm.at[pl.ds(s * _RPS, _RPS)],
                        acc.at[pl.ds(s * _RPS, _RPS)])
        pltpu.sync_copy(src_hbm.at[pl.ds(wid * _GP_TILE, _GP_TILE)], src_v)
        pltpu.sync_copy(dst_hbm.at[pl.ds(wid * _GP_TILE, _GP_TILE)], dst_v)
        plsc.subcore_barrier()

        ring = ((rows0, sem0), (rows1, sem1), (rows2, sem2), (rows3, sem3))
        nbuf = len(ring)
        for b, (rows, sem) in enumerate(ring):
            pltpu.async_copy(h_hbm.at[src_v.at[b]], rows, sem)

        def body(i, carry):
            for b, (rows, sem) in enumerate(ring):
                j = i * nbuf + b
                pltpu.make_async_copy(h_hbm.at[pl.ds(0, _G)], rows, sem).wait()
                pltpu.sync_copy(rows, acc.at[dst_v.at[j]], add=True)

                @pl.when(j + nbuf < _GP_TILE)
                def _():
                    pltpu.async_copy(h_hbm.at[src_v.at[j + nbuf]], rows, sem)

            return carry

        lax.fori_loop(0, _GP_TILE // nbuf, body, 0)
        plsc.subcore_barrier()
        pltpu.sync_copy(acc.at[pl.ds(s * _RPS, _RPS)],
                        out_hbm.at[c, pl.ds(s * _RPS, _RPS)])

    return scatter_kernel


_count_call = _make_count()
_scatter64_call = _make_scatter(_HID)
_scatter32_call = _make_scatter(_DIM)


def _enc1_kernel(x_ref, w0_ref, cnt_ref, h1p_ref, dinv_ref):
    cnt = (jnp.sum(cnt_ref[0], axis=1, keepdims=True)
           + jnp.sum(cnt_ref[1], axis=1, keepdims=True)) * (1.0 / _CNTW)
    dinv = lax.rsqrt(cnt + 1.0)
    xw = jnp.dot(x_ref[...], w0_ref[...],
                 preferred_element_type=jnp.float32)
    h1p_ref[...] = xw * dinv
    dinv_ref[...] = dinv


def _enc2_kernel(p_ref, h1p_ref, dinv_ref, noise_ref, w1_ref, h2p_ref):
    t = (p_ref[0] + p_ref[1] + h1p_ref[...]) * dinv_ref[...]
    hidden = jnp.maximum(t, 0.0) + noise_ref[...]
    h2 = jnp.dot(hidden, w1_ref[...],
                 preferred_element_type=jnp.float32)
    h2p_ref[...] = h2 * dinv_ref[...]


def _dec_prep_kernel(q_ref, h2p_ref, dinv_ref, pr_ref, coords_ref, sq_ref, mass_ref):
    z = (q_ref[0] + q_ref[1] + h2p_ref[...]) * dinv_ref[...]
    mass_ref[...] = z[:, _DIM - 1:_DIM] + pr_ref[...]
    lane = lax.broadcasted_iota(jnp.int32, (_N, _DIM), 1)
    coords = jnp.where(lane < _DIM - 1, z, 0.0)
    nrm = jnp.sqrt(jnp.sum(coords * coords, axis=1, keepdims=True))
    cn = coords / (nrm + _EPS)
    coords_ref[...] = cn
    sq_ref[...] = jnp.sum(cn * cn, axis=1, keepdims=True)


def _dec_kernel(a_ref, call_ref, sqr_ref, sqc_ref, mass_ref, out_ref):
    dot = lax.dot_general(a_ref[...], call_ref[...],
                          (((1,), (1,)), ((), ())),
                          preferred_element_type=jnp.float32)
    d2 = jnp.maximum(sqr_ref[...] + sqc_ref[...] - 2.0 * dot, 0.0)
    out_ref[...] = mass_ref[...] - jnp.log(d2 + _EPS)


def kernel(x, edge_index, pagerank, W0, W1):
    src = edge_index[0].astype(jnp.int32).reshape(_GROUPS, _G)
    dst = edge_index[1].astype(jnp.int32).reshape(_GROUPS, _G)
    ones_rows = jnp.ones((_G, _CNTW), jnp.float32)
    zeros_cnt = jnp.zeros((_N, _CNTW), jnp.float32)
    zeros64 = jnp.zeros((_N, _HID), jnp.float32)
    zeros32 = jnp.zeros((_N, _DIM), jnp.float32)
    noise = _NOISE_STD * jax.random.normal(jax.random.key(42), (_N, _HID),
                                           dtype=jnp.float32)

    cnt = _count_call(dst, ones_rows, zeros_cnt)

    h1p, dinv = pl.pallas_call(
        _enc1_kernel,
        out_shape=[
            jax.ShapeDtypeStruct((_N, _HID), jnp.float32),
            jax.ShapeDtypeStruct((_N, 1), jnp.float32),
        ],
    )(x, W0, cnt)

    p = _scatter64_call(src, dst, h1p, zeros64)

    h2p = pl.pallas_call(
        _enc2_kernel,
        out_shape=jax.ShapeDtypeStruct((_N, _DIM), jnp.float32),
    )(p, h1p, dinv, noise, W1)

    q = _scatter32_call(src, dst, h2p, zeros32)

    coords, sq, mass = pl.pallas_call(
        _dec_prep_kernel,
        out_shape=[
            jax.ShapeDtypeStruct((_N, _DIM), jnp.float32),
            jax.ShapeDtypeStruct((_N, 1), jnp.float32),
            jax.ShapeDtypeStruct((_N, 1), jnp.float32),
        ],
    )(q, h2p, dinv, pagerank.reshape(_N, 1))

    sq_row = sq.reshape(1, _N)
    mass_row = mass.reshape(1, _N)

    logits = pl.pallas_call(
        _dec_kernel,
        grid=(_N // _ROWBLK,),
        in_specs=[
            pl.BlockSpec((_ROWBLK, _DIM), lambda i: (i, 0)),
            pl.BlockSpec((_N, _DIM), lambda i: (0, 0)),
            pl.BlockSpec((_ROWBLK, 1), lambda i: (i, 0)),
            pl.BlockSpec((1, _N), lambda i: (0, 0)),
            pl.BlockSpec((1, _N), lambda i: (0, 0)),
        ],
        out_specs=pl.BlockSpec((_ROWBLK, _N), lambda i: (i, 0)),
        out_shape=jax.ShapeDtypeStruct((_N, _N), jnp.float32),
    )(coords, coords, sq, sq_row, mass_row)
    return logits


# speedup vs baseline: 24.7692x; 1.0007x over previous
import functools

import jax
import jax.numpy as jnp
from jax import lax
from jax.experimental import pallas as pl
from jax.experimental.pallas import tpu as pltpu
from jax.experimental.pallas import tpu_sc as plsc

_N = 10000
_E = 320000
_D_IN = 128
_HID = 64
_DIM = 32
_NOISE_STD = 0.1
_EPS = 1e-6

_G = 125
_GROUPS = _E // _G
_NC = 2
_NS = 16
_NTILES = _NC * _NS
_GP_TILE = _GROUPS // _NTILES
_RPS = _N // _NS
_CNTW = 8

_ROWBLK = 200
_NBUF = 8


def _sc_mesh():
    return plsc.VectorSubcoreMesh(core_axis_name="c", subcore_axis_name="s")


def _make_count():
    @functools.partial(
        pl.kernel,
        out_type=jax.ShapeDtypeStruct((_NC, _N, _CNTW), jnp.float32),
        mesh=_sc_mesh(),
        compiler_params=pltpu.CompilerParams(use_tc_tiling_on_sc=False),
        scratch_types=[
            pltpu.VMEM((_GP_TILE, _G), jnp.int32),
            pltpu.VMEM((_G, _CNTW), jnp.float32),
            pltpu.VMEM_SHARED((_N, _CNTW), jnp.float32),
        ],
    )
    def count_kernel(dst_hbm, ones_hbm, zeros_hbm, out_hbm, dst_v, ones_v, acc):
        c = lax.axis_index("c")
        s = lax.axis_index("s")
        wid = c * _NS + s
        pltpu.sync_copy(zeros_hbm.at[pl.ds(s * _RPS, _RPS)],
                        acc.at[pl.ds(s * _RPS, _RPS)])
        pltpu.sync_copy(dst_hbm.at[pl.ds(wid * _GP_TILE, _GP_TILE)], dst_v)
        pltpu.sync_copy(ones_hbm, ones_v)
        plsc.subcore_barrier()

        def body(j, carry):
            pltpu.sync_copy(ones_v, acc.at[dst_v.at[j]], add=True)
            return carry

        lax.fori_loop(0, _GP_TILE, body, 0)
        plsc.subcore_barrier()
        pltpu.sync_copy(acc.at[pl.ds(s * _RPS, _RPS)],
                        out_hbm.at[c, pl.ds(s * _RPS, _RPS)])

    return count_kernel


def _make_scatter(feat):
    @functools.partial(
        pl.kernel,
        out_type=jax.ShapeDtypeStruct((_NC, _N, feat), jnp.float32),
        mesh=_sc_mesh(),
        compiler_params=pltpu.CompilerParams(use_tc_tiling_on_sc=False),
        scratch_types=[
            pltpu.VMEM((_GP_TILE, _G), jnp.int32),
            pltpu.VMEM((_GP_TILE, _G), jnp.int32),
        ] + [pltpu.VMEM((_G, feat), jnp.float32)] * _NBUF
          + [pltpu.VMEM_SHARED((_N, feat), jnp.float32)]
          + [pltpu.SemaphoreType.DMA] * _NBUF,
    )
    def scatter_kernel(src_hbm, dst_hbm, h_hbm, zeros_hbm, out_hbm,
                       src_v, dst_v, *rest):
        bufs = rest[:_NBUF]
        acc = rest[_NBUF]
        sems = rest[_NBUF + 1:]
        c = lax.axis_index("c")
        s = lax.axis_index("s")
        wid = c * _NS + s
        pltpu.sync_copy(zeros_hbm.at[pl.ds(s * _RPS, _RPS)],
                        acc.at[pl.ds(s * _RPS, _RPS)])
        pltpu.sync_copy(src_hbm.at[pl.ds(wid * _GP_TILE, _GP_TILE)], src_v)
        pltpu.sync_copy(dst_hbm.at[pl.ds(wid * _GP_TILE, _GP_TILE)], dst_v)
        plsc.subcore_barrier()

        ring = tuple(zip(bufs, sems))
        nbuf = len(ring)
        for b, (rows, sem) in enumerate(ring):
            pltpu.async_copy(h_hbm.at[src_v.at[b]], rows, sem)

        def body(i, carry):
            for b, (rows, sem) in enumerate(ring):
                j = i * nbuf + b
                pltpu.make_async_copy(h_hbm.at[pl.ds(0, _G)], rows, sem).wait()
                pltpu.sync_copy(rows, acc.at[dst_v.at[j]], add=True)

                @pl.when(j + nbuf < _GP_TILE)
                def _():
                    pltpu.async_copy(h_hbm.at[src_v.at[j + nbuf]], rows, sem)

            return carry

        lax.fori_loop(0, _GP_TILE // nbuf, body, 0)
        plsc.subcore_barrier()
        pltpu.sync_copy(acc.at[pl.ds(s * _RPS, _RPS)],
                        out_hbm.at[c, pl.ds(s * _RPS, _RPS)])

    return scatter_kernel


_count_call = _make_count()
_scatter64_call = _make_scatter(_HID)
_scatter32_call = _make_scatter(_DIM)


def _enc1_kernel(x_ref, w0_ref, cnt_ref, h1p_ref, dinv_ref):
    cnt = (jnp.sum(cnt_ref[0], axis=1, keepdims=True)
           + jnp.sum(cnt_ref[1], axis=1, keepdims=True)) * (1.0 / _CNTW)
    dinv = lax.rsqrt(cnt + 1.0)
    xw = jnp.dot(x_ref[...], w0_ref[...],
                 preferred_element_type=jnp.float32)
    h1p_ref[...] = xw * dinv
    dinv_ref[...] = dinv


def _enc2_kernel(p_ref, h1p_ref, dinv_ref, noise_ref, w1_ref, h2p_ref):
    t = (p_ref[0] + p_ref[1] + h1p_ref[...]) * dinv_ref[...]
    hidden = jnp.maximum(t, 0.0) + noise_ref[...]
    h2 = jnp.dot(hidden, w1_ref[...],
                 preferred_element_type=jnp.float32)
    h2p_ref[...] = h2 * dinv_ref[...]


def _dec_prep_kernel(q_ref, h2p_ref, dinv_ref, pr_ref, coords_ref, sq_ref, mass_ref):
    z = (q_ref[0] + q_ref[1] + h2p_ref[...]) * dinv_ref[...]
    mass_ref[...] = z[:, _DIM - 1:_DIM] + pr_ref[...]
    lane = lax.broadcasted_iota(jnp.int32, (_N, _DIM), 1)
    coords = jnp.where(lane < _DIM - 1, z, 0.0)
    nrm = jnp.sqrt(jnp.sum(coords * coords, axis=1, keepdims=True))
    cn = coords / (nrm + _EPS)
    coords_ref[...] = cn
    sq_ref[...] = jnp.sum(cn * cn, axis=1, keepdims=True)


def _dec_kernel(a_ref, call_ref, sqr_ref, sqc_ref, mass_ref, out_ref):
    dot = lax.dot_general(a_ref[...], call_ref[...],
                          (((1,), (1,)), ((), ())),
                          preferred_element_type=jnp.float32)
    d2 = jnp.maximum(sqr_ref[...] + sqc_ref[...] - 2.0 * dot, 0.0)
    out_ref[...] = mass_ref[...] - jnp.log(d2 + _EPS)


def kernel(x, edge_index, pagerank, W0, W1):
    src = edge_index[0].astype(jnp.int32).reshape(_GROUPS, _G)
    dst = edge_index[1].astype(jnp.int32).reshape(_GROUPS, _G)
    ones_rows = jnp.ones((_G, _CNTW), jnp.float32)
    zeros_cnt = jnp.zeros((_N, _CNTW), jnp.float32)
    zeros64 = jnp.zeros((_N, _HID), jnp.float32)
    zeros32 = jnp.zeros((_N, _DIM), jnp.float32)
    noise = _NOISE_STD * jax.random.normal(jax.random.key(42), (_N, _HID),
                                           dtype=jnp.float32)

    cnt = _count_call(dst, ones_rows, zeros_cnt)

    h1p, dinv = pl.pallas_call(
        _enc1_kernel,
        out_shape=[
            jax.ShapeDtypeStruct((_N, _HID), jnp.float32),
            jax.ShapeDtypeStruct((_N, 1), jnp.float32),
        ],
    )(x, W0, cnt)

    p = _scatter64_call(src, dst, h1p, zeros64)

    h2p = pl.pallas_call(
        _enc2_kernel,
        out_shape=jax.ShapeDtypeStruct((_N, _DIM), jnp.float32),
    )(p, h1p, dinv, noise, W1)

    q = _scatter32_call(src, dst, h2p, zeros32)

    coords, sq, mass = pl.pallas_call(
        _dec_prep_kernel,
        out_shape=[
            jax.ShapeDtypeStruct((_N, _DIM), jnp.float32),
            jax.ShapeDtypeStruct((_N, 1), jnp.float32),
            jax.ShapeDtypeStruct((_N, 1), jnp.float32),
        ],
    )(q, h2p, dinv, pagerank.reshape(_N, 1))

    sq_row = sq.reshape(1, _N)
    mass_row = mass.reshape(1, _N)

    logits = pl.pallas_call(
        _dec_kernel,
        grid=(_N // _ROWBLK,),
        in_specs=[
            pl.BlockSpec((_ROWBLK, _DIM), lambda i: (i, 0)),
            pl.BlockSpec((_N, _DIM), lambda i: (0, 0)),
            pl.BlockSpec((_ROWBLK, 1), lambda i: (i, 0)),
            pl.BlockSpec((1, _N), lambda i: (0, 0)),
            pl.BlockSpec((1, _N), lambda i: (0, 0)),
        ],
        out_specs=pl.BlockSpec((_ROWBLK, _N), lambda i: (i, 0)),
        out_shape=jax.ShapeDtypeStruct((_N, _N), jnp.float32),
    )(coords, coords, sq, sq_row, mass_row)
    return logits
